# baseline (speedup 1.0000x reference)
import jax
import jax.numpy as jnp
from jax.experimental import pallas as pl

N = 50000
BLK = 2000
HID = 64
L_V = 4
L_C = 8


def _fourier_feats(x, L):
    feats = []
    for i in range(L):
        f = 2.0 ** i
        feats.append(jnp.sin(f * x))
        feats.append(jnp.cos(f * x))
    return jnp.concatenate(feats, axis=-1)


def _make_embed(L, D):
    def body(x_ref, w_ref, b_ref, o_ref):
        feats = _fourier_feats(x_ref[...], L).astype(jnp.bfloat16)
        y = jax.lax.dot_general(feats, w_ref[...], (((1,), (0,)), ((), ())),
                                preferred_element_type=jnp.float32)
        o_ref[...] = y + b_ref[...]

    return pl.pallas_call(
        body,
        grid=(N // BLK,),
        in_specs=[pl.BlockSpec((BLK, D), lambda i: (i, 0)),
                  pl.BlockSpec((HID, HID), lambda i: (0, 0)),
                  pl.BlockSpec((1, HID), lambda i: (0, 0))],
        out_specs=pl.BlockSpec((BLK, HID), lambda i: (i, 0)),
        out_shape=jax.ShapeDtypeStruct((N, HID), jnp.float32),
    )


_embed_v = _make_embed(L_V, 8)
_embed_c = _make_embed(L_C, 4)


def _t_body(i_ref, o_ref):
    o_ref[...] = jnp.transpose(i_ref[...])


_transpose_nm = pl.pallas_call(
    _t_body,
    out_shape=jax.ShapeDtypeStruct((HID, N), jnp.float32),
)


def _graph_norm_t(x, ms, w, b):
    mean = jnp.mean(x, axis=1)
    out = x - (ms * mean)[:, None]
    var = jnp.mean(out * out, axis=1)
    return out / jnp.sqrt(var + 1e-5)[:, None] * w[:, None] + b[:, None]


def _layernorm(x, g, b):
    m = jnp.mean(x, axis=-1, keepdims=True)
    v = jnp.var(x, axis=-1, keepdims=True)
    return (x - m) / jnp.sqrt(v + 1e-5) * g + b


def _mlp(x, p):
    h = x @ p['W1'] + p['b1']
    h = _layernorm(h, p['ln_g'], p['ln_b'])
    h = jax.nn.relu(h)
    return h @ p['W2'] + p['b2']


def kernel(x_vars, x_cons, edge_attr_cv, edge_attr_vc, params, edge_index_cv, edge_index_vc):
    gn = params['gn']
    ms = gn['mean_scale']
    w = gn['weight']
    b = gn['bias']

    xv = _embed_v(x_vars, params['var_W'], params['var_b'].reshape(1, HID))
    xc = _embed_c(x_cons, params['cons_W'], params['cons_b'].reshape(1, HID))
    xvt = _transpose_nm(xv)
    xct = _transpose_nm(xc)

    x = {'vars': xvt, 'cons': xct}
    for _ in range(2):
        x = {k: _graph_norm_t(v, ms, w, b) for k, v in x.items()}
    xvp = jnp.mean(x['vars'], axis=1)[None, :]
    xcp = jnp.mean(x['cons'], axis=1)[None, :]
    xg = jnp.concatenate([xvp, xcp], axis=1)
    pe = _mlp(xg, params['pool_mlp'])
    return _mlp(pe, params['pred_mlp'])
